# baseline (speedup 1.0000x reference)
import functools

import jax
import jax.numpy as jnp
from jax import lax
from jax.experimental import pallas as pl
from jax.experimental.pallas import tpu as pltpu
from jax.experimental.pallas import tpu_sc as plsc

BATCH = 16384
FIELDS = 26
D = 64
NC, NS = 2, 16
NW = NC * NS
TOTAL = BATCH * FIELDS
CHUNK = 128
NCH = TOTAL // (NW * CHUNK)
NBUF = 8
STEADY = NCH - NBUF
assert STEADY % NBUF == 0

_mesh = plsc.VectorSubcoreMesh(core_axis_name="c", subcore_axis_name="s")

_scratch = (
    [pltpu.VMEM((NCH, CHUNK), jnp.int32)]
    + [pltpu.VMEM((CHUNK, D), jnp.float32) for _ in range(NBUF)]
    + [pltpu.SemaphoreType.DMA for _ in range(NBUF)]
)


@functools.partial(
    pl.kernel,
    mesh=_mesh,
    out_type=jax.ShapeDtypeStruct((NW * NCH, CHUNK, D), jnp.float32),
    scratch_types=_scratch,
    compiler_params=pltpu.CompilerParams(use_tc_tiling_on_sc=False),
)
def _gather_k(table_hbm, x_hbm, out_hbm, idx_v, *rest):
    bufs = rest[:NBUF]
    gsems = rest[NBUF:2 * NBUF]
    w = lax.axis_index("s") * NC + lax.axis_index("c")
    base = w * NCH
    pltpu.sync_copy(x_hbm.at[pl.ds(base, NCH)], idx_v)

    for b in range(NBUF):
        pltpu.async_copy(table_hbm.at[idx_v.at[b]], bufs[b], gsems[b])

    def body(i, carry):
        g0 = i * NBUF
        for b in range(NBUF):
            g = g0 + b
            pltpu.make_async_copy(
                table_hbm.at[idx_v.at[0]], bufs[b], gsems[b]).wait()
            pltpu.sync_copy(bufs[b], out_hbm.at[base + g])
            pltpu.async_copy(
                table_hbm.at[idx_v.at[g + NBUF]], bufs[b], gsems[b])
        return carry

    lax.fori_loop(0, STEADY // NBUF, body, 0)

    for b in range(NBUF):
        g = STEADY + b
        pltpu.make_async_copy(
            table_hbm.at[idx_v.at[0]], bufs[b], gsems[b]).wait()
        pltpu.sync_copy(bufs[b], out_hbm.at[base + g])


def kernel(x, table):
    idx = x.astype(jnp.int32).reshape(NW * NCH, CHUNK)
    out = _gather_k(table, idx)
    return out.reshape(BATCH, FIELDS, D)
